# baseline (speedup 1.0000x reference)
import jax
import jax.numpy as jnp
from jax.experimental import pallas as pl
from jax.experimental.pallas import tpu as pltpu

N_F = 1024
N_K = 4096
B_TILE = 512
P_CH = 8
KC = N_K // P_CH


def _mega_kernel(x_ref, a_ref, b_ref, inner_ref, out_ref, an_ref, bb_ref):
    i = pl.program_id(0)

    @pl.when(i < P_CH)
    def _prep():
        a = a_ref[...]
        inv = jax.lax.rsqrt(jnp.sum(a * a, axis=0, keepdims=True))
        an_ref[:, pl.ds(i * KC, KC)] = (a * inv).astype(jnp.bfloat16)
        bb_ref[pl.ds(i * KC, KC), :] = b_ref[...].astype(jnp.bfloat16)

    @pl.when((i >= 1) & (i <= P_CH))
    def _tile0_chunk():
        c = i - 1
        xb = x_ref[...].astype(jnp.bfloat16)
        ic = jnp.dot(xb, an_ref[:, pl.ds(c * KC, KC)],
                     preferred_element_type=jnp.float32)
        inner_ref[:, pl.ds(c * KC, KC)] = ic
        part = jnp.dot(ic.astype(jnp.bfloat16), bb_ref[pl.ds(c * KC, KC), :],
                       preferred_element_type=jnp.float32)

        @pl.when(i == 1)
        def _():
            out_ref[...] = part

        @pl.when(i > 1)
        def _():
            out_ref[...] += part

    @pl.when(i > P_CH)
    def _fused():
        xb = x_ref[...].astype(jnp.bfloat16)
        inner = jnp.dot(xb, an_ref[...], preferred_element_type=jnp.float32)
        inner_ref[...] = inner
        out_ref[...] = jnp.dot(inner.astype(jnp.bfloat16), bb_ref[...],
                               preferred_element_type=jnp.float32)


def kernel(x, A, B):
    batch = x.shape[0]
    nb = batch // B_TILE
    grid = (P_CH + nb,)

    def x_idx(i):
        return (jnp.maximum(i - P_CH, 0), 0)

    def a_idx(i):
        return (0, jnp.minimum(i, P_CH - 1))

    def b_idx(i):
        return (jnp.minimum(i, P_CH - 1), 0)

    inner, out = pl.pallas_call(
        _mega_kernel,
        grid=grid,
        in_specs=[
            pl.BlockSpec((B_TILE, N_F), x_idx),
            pl.BlockSpec((N_F, KC), a_idx),
            pl.BlockSpec((KC, N_F), b_idx),
        ],
        out_specs=[
            pl.BlockSpec((B_TILE, N_K), x_idx),
            pl.BlockSpec((B_TILE, N_F), x_idx),
        ],
        out_shape=[
            jax.ShapeDtypeStruct((batch, N_K), jnp.float32),
            jax.ShapeDtypeStruct((batch, N_F), jnp.float32),
        ],
        scratch_shapes=[
            pltpu.VMEM((N_F, N_K), jnp.bfloat16),
            pltpu.VMEM((N_K, N_F), jnp.bfloat16),
        ],
        compiler_params=pltpu.CompilerParams(
            dimension_semantics=("arbitrary",),
        ),
    )(x, A, B)
    return (out, inner)


# speedup vs baseline: 1.0158x; 1.0158x over previous
import jax
import jax.numpy as jnp
from jax.experimental import pallas as pl
from jax.experimental.pallas import tpu as pltpu

N_F = 1024
N_K = 4096
B_TILE = 512
P_CH = 8
KC = N_K // P_CH


def _mega_kernel(x_ref, a_ref, b_ref, inner_ref, out_ref, an_ref, bb_ref):
    i = pl.program_id(0)

    @pl.when(i < P_CH)
    def _prep():
        a = a_ref[...]
        inv = jax.lax.rsqrt(jnp.sum(a * a, axis=0, keepdims=True))
        an_c = (a * inv).astype(jnp.bfloat16)
        bb_c = b_ref[...].astype(jnp.bfloat16)
        an_ref[:, pl.ds(i * KC, KC)] = an_c
        bb_ref[pl.ds(i * KC, KC), :] = bb_c
        xb = x_ref[...].astype(jnp.bfloat16)
        ic = jnp.dot(xb, an_c, preferred_element_type=jnp.float32)
        inner_ref[:, pl.ds(i * KC, KC)] = ic
        part = jnp.dot(ic.astype(jnp.bfloat16), bb_c,
                       preferred_element_type=jnp.float32)

        @pl.when(i == 0)
        def _():
            out_ref[...] = part

        @pl.when(i > 0)
        def _():
            out_ref[...] += part

    @pl.when(i >= P_CH)
    def _fused():
        xb = x_ref[...].astype(jnp.bfloat16)
        inner = jnp.dot(xb, an_ref[...], preferred_element_type=jnp.float32)
        inner_ref[...] = inner
        out_ref[...] = jnp.dot(inner.astype(jnp.bfloat16), bb_ref[...],
                               preferred_element_type=jnp.float32)


def kernel(x, A, B):
    batch = x.shape[0]
    nb = batch // B_TILE
    grid = (P_CH + nb - 1,)

    def x_idx(i):
        return (jnp.maximum(i - (P_CH - 1), 0), 0)

    def a_idx(i):
        return (0, jnp.minimum(i, P_CH - 1))

    def b_idx(i):
        return (jnp.minimum(i, P_CH - 1), 0)

    inner, out = pl.pallas_call(
        _mega_kernel,
        grid=grid,
        in_specs=[
            pl.BlockSpec((B_TILE, N_F), x_idx),
            pl.BlockSpec((N_F, KC), a_idx),
            pl.BlockSpec((KC, N_F), b_idx),
        ],
        out_specs=[
            pl.BlockSpec((B_TILE, N_K), x_idx),
            pl.BlockSpec((B_TILE, N_F), x_idx),
        ],
        out_shape=[
            jax.ShapeDtypeStruct((batch, N_K), jnp.float32),
            jax.ShapeDtypeStruct((batch, N_F), jnp.float32),
        ],
        scratch_shapes=[
            pltpu.VMEM((N_F, N_K), jnp.bfloat16),
            pltpu.VMEM((N_K, N_F), jnp.bfloat16),
        ],
        compiler_params=pltpu.CompilerParams(
            dimension_semantics=("arbitrary",),
        ),
    )(x, A, B)
    return (out, inner)
